# Initial kernel scaffold; baseline (speedup 1.0000x reference)
import jax
import jax.numpy as jnp
from jax.experimental import pallas as pl


def kernel(index_tensor, embedding_table):
    raise NotImplementedError("write your pallas kernel here")


# speedup vs baseline: 1.8656x; 1.8656x over previous
import functools

import jax
import jax.numpy as jnp
from jax import lax
from jax.experimental import pallas as pl
from jax.experimental.pallas import tpu as pltpu
from jax.experimental.pallas import tpu_sc as plsc

BATCH = 16384
HIST = 50
EMBED_DIM = 64
TOTAL = BATCH * HIST

NUM_CORES = 2
NUM_SUBCORES = 16
NUM_WORKERS = NUM_CORES * NUM_SUBCORES
PER_WORKER = TOTAL // NUM_WORKERS

CHUNK = 1600
NUM_CHUNKS = PER_WORKER // CHUNK


def _gather_body(table_hbm, idx_hbm, out_hbm, idx_v, rows_v, sem):
    wid = lax.axis_index("s") * NUM_CORES + lax.axis_index("c")
    start = wid * PER_WORKER

    def body(g, _):
        base = start + g * CHUNK
        pltpu.sync_copy(idx_hbm.at[pl.ds(base, CHUNK)], idx_v)
        pltpu.async_copy(table_hbm.at[idx_v], rows_v, sem).wait()
        pltpu.sync_copy(rows_v, out_hbm.at[pl.ds(base, CHUNK)])
        return 0

    lax.fori_loop(0, NUM_CHUNKS, body, 0)


@jax.jit
def _sc_gather(table, idx_flat):
    mesh = plsc.VectorSubcoreMesh(core_axis_name="c", subcore_axis_name="s")
    kfn = functools.partial(
        pl.kernel,
        mesh=mesh,
        out_type=jax.ShapeDtypeStruct((TOTAL, EMBED_DIM), jnp.float32),
        scratch_types=[
            pltpu.VMEM((CHUNK,), jnp.int32),
            pltpu.VMEM((CHUNK, EMBED_DIM), jnp.float32),
            pltpu.SemaphoreType.DMA,
        ],
        compiler_params=pltpu.CompilerParams(use_tc_tiling_on_sc=False),
    )(_gather_body)
    return kfn(table, idx_flat)


def kernel(index_tensor, embedding_table):
    idx_flat = index_tensor.reshape(-1).astype(jnp.int32)
    out = _sc_gather(embedding_table, idx_flat)
    return out.reshape(BATCH, HIST, EMBED_DIM)


# speedup vs baseline: 1.8745x; 1.0048x over previous
import functools

import jax
import jax.numpy as jnp
from jax import lax
from jax.experimental import pallas as pl
from jax.experimental.pallas import tpu as pltpu
from jax.experimental.pallas import tpu_sc as plsc

BATCH = 16384
HIST = 50
EMBED_DIM = 64
TOTAL = BATCH * HIST

NUM_CORES = 2
NUM_SUBCORES = 16
NUM_WORKERS = NUM_CORES * NUM_SUBCORES
PER_WORKER = TOTAL // NUM_WORKERS

CHUNK = 800
NUM_CHUNKS = PER_WORKER // CHUNK


def _gather_body(table_hbm, idx_hbm, out_hbm, idx_v, rows_v, gsem, ssem):
    wid = lax.axis_index("s") * NUM_CORES + lax.axis_index("c")
    start = wid * PER_WORKER

    pltpu.sync_copy(idx_hbm.at[pl.ds(start, PER_WORKER)], idx_v)

    def gather_start(g, buf):
        pltpu.async_copy(
            table_hbm.at[idx_v.at[pl.ds(g * CHUNK, CHUNK)]], rows_v.at[buf], gsem
        )

    def store_start(g, buf):
        pltpu.async_copy(
            rows_v.at[buf], out_hbm.at[pl.ds(start + g * CHUNK, CHUNK)], ssem
        )

    def one_wait(sem):
        pltpu.make_async_copy(rows_v.at[0], out_hbm.at[pl.ds(start, CHUNK)], sem).wait()

    gather_start(0, 0)

    def body(g, _):
        buf = lax.rem(g, 2)

        @pl.when(g + 1 < NUM_CHUNKS)
        def _():
            @pl.when(g >= 1)
            def _():
                one_wait(ssem)

            gather_start(g + 1, 1 - buf)

        one_wait(gsem)
        store_start(g, buf)
        return 0

    lax.fori_loop(0, NUM_CHUNKS, body, 0)
    one_wait(ssem)
    one_wait(ssem)


@jax.jit
def _sc_gather(table, idx_flat):
    mesh = plsc.VectorSubcoreMesh(core_axis_name="c", subcore_axis_name="s")
    kfn = functools.partial(
        pl.kernel,
        mesh=mesh,
        out_type=jax.ShapeDtypeStruct((TOTAL, EMBED_DIM), jnp.float32),
        scratch_types=[
            pltpu.VMEM((PER_WORKER,), jnp.int32),
            pltpu.VMEM((2, CHUNK, EMBED_DIM), jnp.float32),
            pltpu.SemaphoreType.DMA,
            pltpu.SemaphoreType.DMA,
        ],
        compiler_params=pltpu.CompilerParams(use_tc_tiling_on_sc=False),
    )(_gather_body)
    return kfn(table, idx_flat)


def kernel(index_tensor, embedding_table):
    idx_flat = index_tensor.reshape(-1).astype(jnp.int32)
    out = _sc_gather(embedding_table, idx_flat)
    return out.reshape(BATCH, HIST, EMBED_DIM)
